# baseline (speedup 1.0000x reference)
import functools

import jax
import jax.numpy as jnp
from jax import lax
from jax.experimental import pallas as pl
from jax.experimental.pallas import tpu as pltpu
from jax.experimental.pallas import tpu_sc as plsc

_VIEWS = 1
_TEMP = 0.4
_IGNORE = 0
_MAXPPC = 150
_CLIP_POS = 1.0
_WEIGHT = 1.0
_BLK = 512
_L = 16
_NSUB = 16

_MC, _IA, _IB, _SA, _SB = range(5)
_NSTAT = 5


def _monokey(bits):
    return bits ^ (lax.shift_right_arithmetic(bits, 31) & 0x7FFFFFFF)


def _sc_select_body(C, M, P, HW0, gt_hbm, fov_hbm, feats_hbm,
                    fsel_hbm, labout_hbm, inc_hbm,
                    gt_v, fov_v, lab_v, val_v, cnt_v, hist_v, grid_sh,
                    grid_v, stat_v, rb_v, inc_v, sh_v, labo_v, dstil_v,
                    d0_v, d1_v, d2_v, d3_v, d4_v, d5_v, rows_v, sem):
    dstN_v = (d0_v, d1_v, d2_v, d3_v, d4_v, d5_v)
    HW = HW0
    nch = P // _L
    wid = lax.axis_index("s")
    base_px = wid * P
    b = base_px // HW
    part = base_px % HW

    for c in range(C):
        pltpu.sync_copy(gt_hbm.at[pl.ds((b * C + c) * HW + part, P)],
                        gt_v.at[pl.ds(c * P, P)])
    pltpu.sync_copy(fov_hbm.at[pl.ds(base_px, P)], fov_v)
    rows_desc = pltpu.async_copy(feats_hbm.at[pl.ds(base_px, P), :],
                                 rows_v, sem)

    def pZ(j, carry):
        sh_v[pl.ds(0, _L)] = jnp.zeros((_L,), jnp.int32)
        hist_v[pl.ds(j * _L, _L)] = jnp.zeros((_L,), jnp.int32)
        return carry

    lax.fori_loop(0, C, pZ, 0)

    def pA(i, carry):
        off = i * _L
        one = jnp.zeros((_L,), jnp.int32) + 1
        zero = jnp.zeros((_L,), jnp.int32)
        best = _monokey(gt_v[pl.ds(off, _L)])
        bidx = jnp.zeros((_L,), jnp.int32)
        for c in range(1, C):
            v = _monokey(gt_v[pl.ds(c * P + off, _L)])
            m = v > best
            bidx = jnp.where(m, c, bidx)
            best = jnp.where(m, v, best)
        fov = fov_v[pl.ds(off, _L)]
        val = (fov == 1) & (bidx != _IGNORE)
        lab_v[pl.ds(off, _L)] = bidx
        val_v[pl.ds(off, _L)] = jnp.where(val, one, zero)
        for c in range(1, C):
            mi = jnp.where((bidx == c) & val, one, zero)
            hist_v[pl.ds(c * _L, _L)] = hist_v[pl.ds(c * _L, _L)] + mi
        return carry

    lax.fori_loop(0, nch, pA, 0)

    def pA2(_, carry):
        lanes = lax.iota(jnp.int32, _L)
        zero = jnp.zeros((_L,), jnp.int32)
        ca = zero
        cb = zero
        for c in range(1, C):
            acc = hist_v[pl.ds(c * _L, _L)]
            s = acc[0]
            for q in range(1, _L):
                s = s + acc[q]
            if c < _L:
                ca = jnp.where(lanes == c, zero + s, ca)
            else:
                cb = jnp.where(lanes == (c - _L), zero + s, cb)
        cnt_v[pl.ds(0, _L)] = ca
        cnt_v[pl.ds(_L, _L)] = cb
        return carry

    lax.fori_loop(0, 1, pA2, 0)

    pltpu.sync_copy(cnt_v, grid_sh.at[pl.ds(wid * 2 * _L, 2 * _L)])
    plsc.subcore_barrier()
    pltpu.sync_copy(grid_sh, grid_v)

    def pB(_, carry):
        lanes = lax.iota(jnp.int32, _L)
        zero = jnp.zeros((_L,), jnp.int32)
        ga = zero
        gb = zero
        pa = zero
        pb = zero
        for t in range(_NSUB):
            ra = grid_v[pl.ds(t * 2 * _L, _L)]
            rb = grid_v[pl.ds(t * 2 * _L + _L, _L)]
            ga = ga + ra
            gb = gb + rb
            cond = t < wid
            pa = pa + jnp.where(cond, ra, zero)
            pb = pb + jnp.where(cond, rb, zero)
        rb_v[pl.ds(0, _L)] = pa
        rb_v[pl.ds(_L, _L)] = pb

        g = [None] * C
        for c in range(1, C):
            g[c] = ga[c] if c < _L else gb[c - _L]

        kk = jnp.int32(0)
        for c in range(1, C):
            kk = kk + jnp.where(g[c] > 0, 1, 0)
        lo_t = lax.shift_right_arithmetic(kk - 1, 1)
        hi_t = lax.shift_right_arithmetic(kk, 1)
        lo = jnp.int32(0)
        hi = jnp.int32(0)
        for c in range(1, C):
            nzc = g[c] > 0
            rank = jnp.int32(0)
            for d in range(1, C):
                if d == c:
                    continue
                less = g[d] < g[c]
                tie = (g[d] == g[c]) & (d < c)
                rank = rank + jnp.where((g[d] > 0) & (less | tie), 1, 0)
            lo = jnp.where(nzc & (rank == lo_t), g[c], lo)
            hi = jnp.where(nzc & (rank == hi_t), g[c], hi)
        mcs = jnp.maximum(lax.shift_right_arithmetic(lo + hi, 1), _MAXPPC)

        inca = zero
        incb = zero
        sega = zero
        segb = zero
        run = jnp.int32(0)
        for c in range(1, C):
            selc = jnp.minimum(g[c], mcs)
            segc = run
            run = run + selc
            if c < _L:
                sega = jnp.where(lanes == c, zero + segc, sega)
                inca = jnp.where(lanes == c, zero + run, inca)
            else:
                segb = jnp.where(lanes == (c - _L), zero + segc, segb)
                incb = jnp.where(lanes == (c - _L), zero + run, incb)
        incb = jnp.where(lanes > (C - 1 - _L), zero + run, incb)
        stat_v[pl.ds(_MC * _L, _L)] = zero + mcs
        stat_v[pl.ds(_IA * _L, _L)] = inca
        stat_v[pl.ds(_IB * _L, _L)] = incb
        stat_v[pl.ds(_SA * _L, _L)] = sega
        stat_v[pl.ds(_SB * _L, _L)] = segb
        inc_v[pl.ds(0, _L)] = inca
        inc_v[pl.ds(_L, _L)] = incb
        return carry

    lax.fori_loop(0, 1, pB, 0)

    @pl.when(wid == 0)
    def _():
        pltpu.sync_copy(inc_v, inc_hbm)

    def pC(i, carry):
        off = i * _L
        lanes = lax.iota(jnp.int32, _L)
        one = jnp.zeros((_L,), jnp.int32) + 1
        zero = jnp.zeros((_L,), jnp.int32)
        labv = lab_v[pl.ds(off, _L)]
        valv = val_v[pl.ds(off, _L)] == 1
        rba = rb_v[pl.ds(0, _L)]
        rbb = rb_v[pl.ds(_L, _L)]
        mcv = stat_v[pl.ds(_MC * _L, _L)]
        inca = stat_v[pl.ds(_IA * _L, _L)]
        incb = stat_v[pl.ds(_IB * _L, _L)]
        sega = stat_v[pl.ds(_SA * _L, _L)]
        segb = stat_v[pl.ds(_SB * _L, _L)]
        dst = zero + M
        labacc = one
        jvec = lanes + (base_px + off)
        for c in range(1, C):
            ln = c % _L
            m = (labv == c) & valv
            mi = jnp.where(m, one, zero)
            x = mi
            for k in (1, 2, 4, 8):
                sh_v[pl.ds(_L, _L)] = x
                x = x + sh_v[pl.ds(_L - k, _L)]
            tot = x[_L - 1]
            prior = x - mi
            rbase = rba[ln] if c < _L else rbb[ln]
            r = prior + rbase
            selm = m & (r < mcv)
            segc = sega[ln] if c < _L else segb[ln]
            dst = jnp.where(selm, r + segc, dst)
            if c < _L:
                rba = jnp.where(lanes == ln, zero + (rbase + tot), rba)
            else:
                rbb = jnp.where(lanes == ln, zero + (rbase + tot), rbb)
            incc = inca[ln] if c < _L else incb[ln]
            labacc = labacc + jnp.where(jvec >= incc, one, zero)
        rb_v[pl.ds(0, _L)] = rba
        rb_v[pl.ds(_L, _L)] = rbb
        labo_v[pl.ds(off, _L)] = labacc
        dstil_v[pl.ds(off, _L)] = dst
        return carry

    lax.fori_loop(0, nch, pC, 0)
    pltpu.sync_copy(labo_v, labout_hbm.at[pl.ds(base_px, P)])

    def pD(q, carry):
        for j in range(6):
            dstN_v[j][pl.ds(q * _L, _L)] = (
                dstil_v[pl.ds(j * 96 + q * _L, _L)])
        return carry

    lax.fori_loop(0, 6, pD, 0)

    rows_desc.wait()
    descs = []
    for j in range(P // 96):
        descs.append(pltpu.async_copy(
            rows_v.at[pl.ds(j * 96, 96)], fsel_hbm.at[dstN_v[j]], sem))
    for d in descs:
        d.wait()


def _sc_select(gt1d, fov_i, feats_fl, C, M, HW):
    P = M // _NSUB
    Z = feats_fl.shape[1]
    mesh = plsc.VectorSubcoreMesh(core_axis_name="c", subcore_axis_name="s",
                                  num_cores=1)
    body = functools.partial(_sc_select_body, C, M, P, HW)
    return pl.kernel(
        body,
        out_type=(
            jax.ShapeDtypeStruct((M + 8, Z), jnp.float32),
            jax.ShapeDtypeStruct((M,), jnp.int32),
            jax.ShapeDtypeStruct((2 * _L,), jnp.int32),
        ),
        mesh=mesh,
        scratch_types=[
            pltpu.VMEM((C * P,), jnp.int32),
            pltpu.VMEM((P,), jnp.int32),
            pltpu.VMEM((P,), jnp.int32),
            pltpu.VMEM((P,), jnp.int32),
            pltpu.VMEM((2 * _L,), jnp.int32),
            pltpu.VMEM((20 * _L,), jnp.int32),
            pltpu.VMEM_SHARED((_NSUB * 2 * _L,), jnp.int32),
            pltpu.VMEM((_NSUB * 2 * _L,), jnp.int32),
            pltpu.VMEM((_NSTAT * _L,), jnp.int32),
            pltpu.VMEM((2 * _L,), jnp.int32),
            pltpu.VMEM((2 * _L,), jnp.int32),
            pltpu.VMEM((2 * _L,), jnp.int32),
            pltpu.VMEM((P,), jnp.int32),
            pltpu.VMEM((P,), jnp.int32),
            pltpu.VMEM((96,), jnp.int32),
            pltpu.VMEM((96,), jnp.int32),
            pltpu.VMEM((96,), jnp.int32),
            pltpu.VMEM((96,), jnp.int32),
            pltpu.VMEM((96,), jnp.int32),
            pltpu.VMEM((96,), jnp.int32),
            pltpu.VMEM((P, Z), jnp.float32),
            pltpu.SemaphoreType.DMA,
        ],
    )(gt1d, fov_i, feats_fl)


def _supcon_body(s_ref, labr_ref, labc_ref, f_ref, out_ref, fn_ref, d_ref):
    S = s_ref[0]
    nb = (S + _BLK - 1) // _BLK
    inv_t = 1.0 / _TEMP
    shift = 1.0 / _TEMP

    def norm_body(rb, _):
        blk = f_ref[pl.ds(rb * _BLK, _BLK), :]
        n = jnp.sqrt(jnp.sum(blk * blk, axis=1, keepdims=True))
        fn_ref[pl.ds(rb * _BLK, _BLK), :] = blk / (n + 1e-12)
        return 0

    jax.lax.fori_loop(0, nb, norm_body, 0)

    def d_body(rb, _):
        labr = labr_ref[pl.ds(rb * _BLK, _BLK), :]
        a = fn_ref[pl.ds(rb * _BLK, _BLK), :]

        def cb_body(cb, acc):
            bblk = fn_ref[pl.ds(cb * _BLK, _BLK), :]
            s = jax.lax.dot_general(a, bblk, (((1,), (1,)), ((), ())),
                                    preferred_element_type=jnp.float32)
            l = s * inv_t - shift
            labc = labc_ref[:, pl.ds(cb * _BLK, _BLK)]
            colidx = jax.lax.broadcasted_iota(jnp.int32, (1, _BLK), 1) + cb * _BLK
            negm = (labr != labc) & (colidx < S)
            return acc + jnp.sum(jnp.where(negm, jnp.exp(l), 0.0), axis=1,
                                 keepdims=True)

        acc = jax.lax.fori_loop(0, nb, cb_body,
                                jnp.zeros((_BLK, 1), jnp.float32))
        d_ref[pl.ds(rb * _BLK, _BLK), :] = acc
        return 0

    jax.lax.fori_loop(0, nb, d_body, 0)

    def p_body(rb, carry):
        tot_p, tot_c = carry
        labr = labr_ref[pl.ds(rb * _BLK, _BLK), :]
        a = fn_ref[pl.ds(rb * _BLK, _BLK), :]
        dvec = d_ref[pl.ds(rb * _BLK, _BLK), :]
        rowidx = jax.lax.broadcasted_iota(jnp.int32, (_BLK, 1), 0) + rb * _BLK

        def cb_body(cb, carry2):
            psum, pcnt = carry2
            bblk = fn_ref[pl.ds(cb * _BLK, _BLK), :]
            s = jax.lax.dot_general(a, bblk, (((1,), (1,)), ((), ())),
                                    preferred_element_type=jnp.float32)
            labc = labc_ref[:, pl.ds(cb * _BLK, _BLK)]
            colidx = jax.lax.broadcasted_iota(jnp.int32, (_BLK, _BLK), 1) + cb * _BLK
            posm = (labr == labc) & (rowidx != colidx) & (colidx < S)
            l = jnp.minimum(s, _CLIP_POS) * inv_t - shift
            lp = l - jnp.log(jnp.exp(l) + dvec + 1e-12)
            psum = psum + jnp.sum(jnp.where(posm, lp, 0.0), axis=1,
                                  keepdims=True)
            pcnt = pcnt + jnp.sum(posm.astype(jnp.int32), axis=1,
                                  keepdims=True)
            return psum, pcnt

        psum, pcnt = jax.lax.fori_loop(
            0, nb, cb_body,
            (jnp.zeros((_BLK, 1), jnp.float32),
             jnp.zeros((_BLK, 1), jnp.int32)))
        haspos = pcnt > 0
        mlpp = psum / jnp.maximum(pcnt, 1).astype(jnp.float32)
        tot_p = tot_p + jnp.sum(jnp.where(haspos, mlpp, 0.0))
        tot_c = tot_c + jnp.sum(haspos.astype(jnp.int32))
        return tot_p, tot_c

    tot_p, tot_c = jax.lax.fori_loop(0, nb, p_body,
                                     (jnp.float32(0.0), jnp.int32(0)))
    loss = -tot_p / jnp.maximum(tot_c, 1).astype(jnp.float32)
    out_ref[0, 0] = _WEIGHT * loss


def _supcon_loss(f_sel, lab_sel, S, M):
    labr = lab_sel.reshape(M, 1)
    labc = lab_sel.reshape(1, M)
    out = pl.pallas_call(
        _supcon_body,
        out_shape=jax.ShapeDtypeStruct((1, 1), jnp.float32),
        in_specs=[
            pl.BlockSpec(memory_space=pltpu.SMEM),
            pl.BlockSpec(memory_space=pltpu.VMEM),
            pl.BlockSpec(memory_space=pltpu.VMEM),
            pl.BlockSpec(memory_space=pltpu.VMEM),
        ],
        out_specs=pl.BlockSpec(memory_space=pltpu.SMEM),
        scratch_shapes=[
            pltpu.VMEM((M, 128), jnp.float32),
            pltpu.VMEM((M, 1), jnp.float32),
        ],
    )(S, labr, labc, f_sel)
    return out[0, 0]


def kernel(feats, gt_prob, fov_mask):
    BV, Z, H, W = feats.shape
    B = BV // _VIEWS
    C = gt_prob.shape[1]
    M = B * H * W
    gt1d = lax.bitcast_convert_type(gt_prob, jnp.int32).reshape(BV * C * H * W)
    fov_i = fov_mask.reshape(M).astype(jnp.int32)
    feats_fl = feats.transpose(0, 2, 3, 1).reshape(M, Z)
    f_sel, lab_sel, inc = _sc_select(gt1d, fov_i, feats_fl, C, M, H * W)
    S = inc[C - 1:C]
    return _supcon_loss(f_sel, lab_sel, S, M)


# speedup vs baseline: 1.0612x; 1.0612x over previous
import functools

import jax
import jax.numpy as jnp
from jax import lax
from jax.experimental import pallas as pl
from jax.experimental.pallas import tpu as pltpu
from jax.experimental.pallas import tpu_sc as plsc

_VIEWS = 1
_TEMP = 0.4
_IGNORE = 0
_MAXPPC = 150
_CLIP_POS = 1.0
_WEIGHT = 1.0
_BLK = 1024
_L = 16
_NSUB = 16

_MC, _IA, _IB, _SA, _SB = range(5)
_NSTAT = 5


def _monokey(bits):
    return bits ^ (lax.shift_right_arithmetic(bits, 31) & 0x7FFFFFFF)


def _sc_select_body(C, M, P, HW0, gt_hbm, fov_hbm, feats_hbm,
                    fsel_hbm, labout_hbm, inc_hbm,
                    gt_v, fov_v, lab_v, val_v, cnt_v, hist_v, grid_sh,
                    grid_v, stat_v, rb_v, inc_v, sh_v, labo_v, dstil_v,
                    d0_v, d1_v, d2_v, d3_v, d4_v, d5_v, rows_v, sem, gsem):
    dstN_v = (d0_v, d1_v, d2_v, d3_v, d4_v, d5_v)
    HW = HW0
    nch = P // _L
    wid = lax.axis_index("s")
    base_px = wid * P
    b = base_px // HW
    part = base_px % HW

    gt_descs = []
    for c in range(C):
        gt_descs.append(pltpu.async_copy(
            gt_hbm.at[pl.ds((b * C + c) * HW + part, P)],
            gt_v.at[pl.ds(c * P, P)], gsem))
    gt_descs.append(pltpu.async_copy(fov_hbm.at[pl.ds(base_px, P)],
                                     fov_v, gsem))
    rows_desc = pltpu.async_copy(feats_hbm.at[pl.ds(base_px, P), :],
                                 rows_v, sem)

    def pZ(j, carry):
        sh_v[pl.ds(0, _L)] = jnp.zeros((_L,), jnp.int32)
        hist_v[pl.ds(j * _L, _L)] = jnp.zeros((_L,), jnp.int32)
        return carry

    lax.fori_loop(0, C, pZ, 0)
    for d in gt_descs:
        d.wait()

    def pA(i, carry):
        off = i * _L
        one = jnp.zeros((_L,), jnp.int32) + 1
        zero = jnp.zeros((_L,), jnp.int32)
        best = _monokey(gt_v[pl.ds(off, _L)])
        bidx = jnp.zeros((_L,), jnp.int32)
        for c in range(1, C):
            v = _monokey(gt_v[pl.ds(c * P + off, _L)])
            m = v > best
            bidx = jnp.where(m, c, bidx)
            best = jnp.where(m, v, best)
        fov = fov_v[pl.ds(off, _L)]
        val = (fov == 1) & (bidx != _IGNORE)
        lab_v[pl.ds(off, _L)] = bidx
        val_v[pl.ds(off, _L)] = jnp.where(val, one, zero)
        for c in range(1, C):
            mi = jnp.where((bidx == c) & val, one, zero)
            hist_v[pl.ds(c * _L, _L)] = hist_v[pl.ds(c * _L, _L)] + mi
        return carry

    lax.fori_loop(0, nch, pA, 0)

    def pA2(_, carry):
        lanes = lax.iota(jnp.int32, _L)
        zero = jnp.zeros((_L,), jnp.int32)
        ca = zero
        cb = zero
        for c in range(1, C):
            acc = hist_v[pl.ds(c * _L, _L)]
            s = acc[0]
            for q in range(1, _L):
                s = s + acc[q]
            if c < _L:
                ca = jnp.where(lanes == c, zero + s, ca)
            else:
                cb = jnp.where(lanes == (c - _L), zero + s, cb)
        cnt_v[pl.ds(0, _L)] = ca
        cnt_v[pl.ds(_L, _L)] = cb
        return carry

    lax.fori_loop(0, 1, pA2, 0)

    pltpu.sync_copy(cnt_v, grid_sh.at[pl.ds(wid * 2 * _L, 2 * _L)])
    plsc.subcore_barrier()
    pltpu.sync_copy(grid_sh, grid_v)

    def pB(_, carry):
        lanes = lax.iota(jnp.int32, _L)
        zero = jnp.zeros((_L,), jnp.int32)
        ga = zero
        gb = zero
        pa = zero
        pb = zero
        for t in range(_NSUB):
            ra = grid_v[pl.ds(t * 2 * _L, _L)]
            rb = grid_v[pl.ds(t * 2 * _L + _L, _L)]
            ga = ga + ra
            gb = gb + rb
            cond = t < wid
            pa = pa + jnp.where(cond, ra, zero)
            pb = pb + jnp.where(cond, rb, zero)
        rb_v[pl.ds(0, _L)] = pa
        rb_v[pl.ds(_L, _L)] = pb

        g = [None] * C
        for c in range(1, C):
            g[c] = ga[c] if c < _L else gb[c - _L]

        kk = jnp.int32(0)
        for c in range(1, C):
            kk = kk + jnp.where(g[c] > 0, 1, 0)
        lo_t = lax.shift_right_arithmetic(kk - 1, 1)
        hi_t = lax.shift_right_arithmetic(kk, 1)
        lo = jnp.int32(0)
        hi = jnp.int32(0)
        for c in range(1, C):
            nzc = g[c] > 0
            rank = jnp.int32(0)
            for d in range(1, C):
                if d == c:
                    continue
                less = g[d] < g[c]
                tie = (g[d] == g[c]) & (d < c)
                rank = rank + jnp.where((g[d] > 0) & (less | tie), 1, 0)
            lo = jnp.where(nzc & (rank == lo_t), g[c], lo)
            hi = jnp.where(nzc & (rank == hi_t), g[c], hi)
        mcs = jnp.maximum(lax.shift_right_arithmetic(lo + hi, 1), _MAXPPC)

        inca = zero
        incb = zero
        sega = zero
        segb = zero
        run = jnp.int32(0)
        for c in range(1, C):
            selc = jnp.minimum(g[c], mcs)
            segc = run
            run = run + selc
            if c < _L:
                sega = jnp.where(lanes == c, zero + segc, sega)
                inca = jnp.where(lanes == c, zero + run, inca)
            else:
                segb = jnp.where(lanes == (c - _L), zero + segc, segb)
                incb = jnp.where(lanes == (c - _L), zero + run, incb)
        incb = jnp.where(lanes > (C - 1 - _L), zero + run, incb)
        stat_v[pl.ds(_MC * _L, _L)] = zero + mcs
        stat_v[pl.ds(_IA * _L, _L)] = inca
        stat_v[pl.ds(_IB * _L, _L)] = incb
        stat_v[pl.ds(_SA * _L, _L)] = sega
        stat_v[pl.ds(_SB * _L, _L)] = segb
        inc_v[pl.ds(0, _L)] = inca
        inc_v[pl.ds(_L, _L)] = incb
        return carry

    lax.fori_loop(0, 1, pB, 0)

    @pl.when(wid == 0)
    def _():
        pltpu.sync_copy(inc_v, inc_hbm)

    def pC(i, carry):
        off = i * _L
        lanes = lax.iota(jnp.int32, _L)
        one = jnp.zeros((_L,), jnp.int32) + 1
        zero = jnp.zeros((_L,), jnp.int32)
        labv = lab_v[pl.ds(off, _L)]
        valv = val_v[pl.ds(off, _L)] == 1
        rba = rb_v[pl.ds(0, _L)]
        rbb = rb_v[pl.ds(_L, _L)]
        mcv = stat_v[pl.ds(_MC * _L, _L)]
        inca = stat_v[pl.ds(_IA * _L, _L)]
        incb = stat_v[pl.ds(_IB * _L, _L)]
        sega = stat_v[pl.ds(_SA * _L, _L)]
        segb = stat_v[pl.ds(_SB * _L, _L)]
        dst = zero + M
        labacc = one
        jvec = lanes + (base_px + off)
        for c in range(1, C):
            ln = c % _L
            m = (labv == c) & valv
            mi = jnp.where(m, one, zero)
            x = mi
            for k in (1, 2, 4, 8):
                sh_v[pl.ds(_L, _L)] = x
                x = x + sh_v[pl.ds(_L - k, _L)]
            tot = x[_L - 1]
            prior = x - mi
            rbase = rba[ln] if c < _L else rbb[ln]
            r = prior + rbase
            selm = m & (r < mcv)
            segc = sega[ln] if c < _L else segb[ln]
            dst = jnp.where(selm, r + segc, dst)
            if c < _L:
                rba = jnp.where(lanes == ln, zero + (rbase + tot), rba)
            else:
                rbb = jnp.where(lanes == ln, zero + (rbase + tot), rbb)
            incc = inca[ln] if c < _L else incb[ln]
            labacc = labacc + jnp.where(jvec >= incc, one, zero)
        rb_v[pl.ds(0, _L)] = rba
        rb_v[pl.ds(_L, _L)] = rbb
        labo_v[pl.ds(off, _L)] = labacc
        dstil_v[pl.ds(off, _L)] = dst
        return carry

    lax.fori_loop(0, nch, pC, 0)
    pltpu.sync_copy(labo_v, labout_hbm.at[pl.ds(base_px, P)])

    def pD(q, carry):
        for j in range(6):
            dstN_v[j][pl.ds(q * _L, _L)] = (
                dstil_v[pl.ds(j * 96 + q * _L, _L)])
        return carry

    lax.fori_loop(0, 6, pD, 0)

    rows_desc.wait()
    descs = []
    for j in range(P // 96):
        descs.append(pltpu.async_copy(
            rows_v.at[pl.ds(j * 96, 96)], fsel_hbm.at[dstN_v[j]], sem))
    for d in descs:
        d.wait()


def _sc_select(gt1d, fov_i, feats_fl, C, M, HW):
    P = M // _NSUB
    Z = feats_fl.shape[1]
    mesh = plsc.VectorSubcoreMesh(core_axis_name="c", subcore_axis_name="s",
                                  num_cores=1)
    body = functools.partial(_sc_select_body, C, M, P, HW)
    return pl.kernel(
        body,
        out_type=(
            jax.ShapeDtypeStruct((M + 8, Z), jnp.float32),
            jax.ShapeDtypeStruct((M,), jnp.int32),
            jax.ShapeDtypeStruct((2 * _L,), jnp.int32),
        ),
        mesh=mesh,
        scratch_types=[
            pltpu.VMEM((C * P,), jnp.int32),
            pltpu.VMEM((P,), jnp.int32),
            pltpu.VMEM((P,), jnp.int32),
            pltpu.VMEM((P,), jnp.int32),
            pltpu.VMEM((2 * _L,), jnp.int32),
            pltpu.VMEM((20 * _L,), jnp.int32),
            pltpu.VMEM_SHARED((_NSUB * 2 * _L,), jnp.int32),
            pltpu.VMEM((_NSUB * 2 * _L,), jnp.int32),
            pltpu.VMEM((_NSTAT * _L,), jnp.int32),
            pltpu.VMEM((2 * _L,), jnp.int32),
            pltpu.VMEM((2 * _L,), jnp.int32),
            pltpu.VMEM((2 * _L,), jnp.int32),
            pltpu.VMEM((P,), jnp.int32),
            pltpu.VMEM((P,), jnp.int32),
            pltpu.VMEM((96,), jnp.int32),
            pltpu.VMEM((96,), jnp.int32),
            pltpu.VMEM((96,), jnp.int32),
            pltpu.VMEM((96,), jnp.int32),
            pltpu.VMEM((96,), jnp.int32),
            pltpu.VMEM((96,), jnp.int32),
            pltpu.VMEM((P, Z), jnp.float32),
            pltpu.SemaphoreType.DMA,
            pltpu.SemaphoreType.DMA,
        ],
    )(gt1d, fov_i, feats_fl)


def _supcon_body(s_ref, labr_ref, labc_ref, f_ref, out_ref, fn_ref, d_ref):
    S = s_ref[0]
    nb = (S + _BLK - 1) // _BLK
    inv_t = 1.0 / _TEMP
    shift = 1.0 / _TEMP

    def norm_body(rb, _):
        blk = f_ref[pl.ds(rb * _BLK, _BLK), :]
        n = jnp.sqrt(jnp.sum(blk * blk, axis=1, keepdims=True))
        fn_ref[pl.ds(rb * _BLK, _BLK), :] = blk / (n + 1e-12)
        return 0

    jax.lax.fori_loop(0, nb, norm_body, 0)

    def d_body(rb, _):
        labr = labr_ref[pl.ds(rb * _BLK, _BLK), :]
        a = fn_ref[pl.ds(rb * _BLK, _BLK), :]

        def cb_body(cb, acc):
            bblk = fn_ref[pl.ds(cb * _BLK, _BLK), :]
            s = jax.lax.dot_general(a, bblk, (((1,), (1,)), ((), ())),
                                    preferred_element_type=jnp.float32)
            l = s * inv_t - shift
            labc = labc_ref[:, pl.ds(cb * _BLK, _BLK)]
            colidx = jax.lax.broadcasted_iota(jnp.int32, (1, _BLK), 1) + cb * _BLK
            negm = (labr != labc) & (colidx < S)
            return acc + jnp.sum(jnp.where(negm, jnp.exp(l), 0.0), axis=1,
                                 keepdims=True)

        acc = jax.lax.fori_loop(0, nb, cb_body,
                                jnp.zeros((_BLK, 1), jnp.float32))
        d_ref[pl.ds(rb * _BLK, _BLK), :] = acc
        return 0

    jax.lax.fori_loop(0, nb, d_body, 0)

    def p_body(rb, carry):
        tot_p, tot_c = carry
        labr = labr_ref[pl.ds(rb * _BLK, _BLK), :]
        a = fn_ref[pl.ds(rb * _BLK, _BLK), :]
        dvec = d_ref[pl.ds(rb * _BLK, _BLK), :]
        rowidx = jax.lax.broadcasted_iota(jnp.int32, (_BLK, 1), 0) + rb * _BLK

        def cb_body(cb, carry2):
            psum, pcnt = carry2
            bblk = fn_ref[pl.ds(cb * _BLK, _BLK), :]
            s = jax.lax.dot_general(a, bblk, (((1,), (1,)), ((), ())),
                                    preferred_element_type=jnp.float32)
            labc = labc_ref[:, pl.ds(cb * _BLK, _BLK)]
            colidx = jax.lax.broadcasted_iota(jnp.int32, (_BLK, _BLK), 1) + cb * _BLK
            posm = (labr == labc) & (rowidx != colidx) & (colidx < S)
            l = jnp.minimum(s, _CLIP_POS) * inv_t - shift
            lp = l - jnp.log(jnp.exp(l) + dvec + 1e-12)
            psum = psum + jnp.sum(jnp.where(posm, lp, 0.0), axis=1,
                                  keepdims=True)
            pcnt = pcnt + jnp.sum(posm.astype(jnp.int32), axis=1,
                                  keepdims=True)
            return psum, pcnt

        psum, pcnt = jax.lax.fori_loop(
            0, nb, cb_body,
            (jnp.zeros((_BLK, 1), jnp.float32),
             jnp.zeros((_BLK, 1), jnp.int32)))
        haspos = pcnt > 0
        mlpp = psum / jnp.maximum(pcnt, 1).astype(jnp.float32)
        tot_p = tot_p + jnp.sum(jnp.where(haspos, mlpp, 0.0))
        tot_c = tot_c + jnp.sum(haspos.astype(jnp.int32))
        return tot_p, tot_c

    tot_p, tot_c = jax.lax.fori_loop(0, nb, p_body,
                                     (jnp.float32(0.0), jnp.int32(0)))
    loss = -tot_p / jnp.maximum(tot_c, 1).astype(jnp.float32)
    out_ref[0, 0] = _WEIGHT * loss


def _supcon_loss(f_sel, lab_sel, S, M):
    labr = lab_sel.reshape(M, 1)
    labc = lab_sel.reshape(1, M)
    out = pl.pallas_call(
        _supcon_body,
        out_shape=jax.ShapeDtypeStruct((1, 1), jnp.float32),
        in_specs=[
            pl.BlockSpec(memory_space=pltpu.SMEM),
            pl.BlockSpec(memory_space=pltpu.VMEM),
            pl.BlockSpec(memory_space=pltpu.VMEM),
            pl.BlockSpec(memory_space=pltpu.VMEM),
        ],
        out_specs=pl.BlockSpec(memory_space=pltpu.SMEM),
        scratch_shapes=[
            pltpu.VMEM((M, 128), jnp.float32),
            pltpu.VMEM((M, 1), jnp.float32),
        ],
    )(S, labr, labc, f_sel)
    return out[0, 0]


def kernel(feats, gt_prob, fov_mask):
    BV, Z, H, W = feats.shape
    B = BV // _VIEWS
    C = gt_prob.shape[1]
    M = B * H * W
    gt1d = lax.bitcast_convert_type(gt_prob, jnp.int32).reshape(BV * C * H * W)
    fov_i = fov_mask.reshape(M).astype(jnp.int32)
    feats_fl = feats.transpose(0, 2, 3, 1).reshape(M, Z)
    f_sel, lab_sel, inc = _sc_select(gt1d, fov_i, feats_fl, C, M, H * W)
    S = inc[C - 1:C]
    return _supcon_loss(f_sel, lab_sel, S, M)


# speedup vs baseline: 1.0620x; 1.0008x over previous
import functools

import jax
import jax.numpy as jnp
from jax import lax
from jax.experimental import pallas as pl
from jax.experimental.pallas import tpu as pltpu
from jax.experimental.pallas import tpu_sc as plsc

_VIEWS = 1
_TEMP = 0.4
_IGNORE = 0
_MAXPPC = 150
_CLIP_POS = 1.0
_WEIGHT = 1.0
_BLK = 1024
_L = 16
_NSUB = 16

_MC, _IA, _IB, _SA, _SB = range(5)
_NSTAT = 5


def _monokey(bits):
    return bits ^ (lax.shift_right_arithmetic(bits, 31) & 0x7FFFFFFF)


def _sc_select_body(C, M, P, HW0, gt_hbm, fov_hbm, feats_hbm,
                    fsel_hbm, labout_hbm, inc_hbm,
                    gt_v, fov_v, lab_v, val_v, cnt_v, hist_v, grid_sh,
                    grid_v, stat_v, rbtab_v, segtab_v, inctab_v,
                    inc_v, sh_v, labo_v, dstil_v,
                    d0_v, d1_v, d2_v, d3_v, d4_v, d5_v, rows_v, sem, gsem):
    dstN_v = (d0_v, d1_v, d2_v, d3_v, d4_v, d5_v)
    HW = HW0
    nch = P // _L
    wid = lax.axis_index("s")
    base_px = wid * P
    b = base_px // HW
    part = base_px % HW

    gt_descs = []
    for c in range(C):
        gt_descs.append(pltpu.async_copy(
            gt_hbm.at[pl.ds((b * C + c) * HW + part, P)],
            gt_v.at[pl.ds(c * P, P)], gsem))
    gt_descs.append(pltpu.async_copy(fov_hbm.at[pl.ds(base_px, P)],
                                     fov_v, gsem))
    rows_desc = pltpu.async_copy(feats_hbm.at[pl.ds(base_px, P), :],
                                 rows_v, sem)

    def pZ(j, carry):
        sh_v[pl.ds(0, _L)] = jnp.zeros((_L,), jnp.int32)
        hist_v[pl.ds(j * _L, _L)] = jnp.zeros((_L,), jnp.int32)
        return carry

    lax.fori_loop(0, C, pZ, 0)
    for d in gt_descs:
        d.wait()

    def pA(i, carry):
        off = i * _L
        one = jnp.zeros((_L,), jnp.int32) + 1
        zero = jnp.zeros((_L,), jnp.int32)
        best = _monokey(gt_v[pl.ds(off, _L)])
        bidx = jnp.zeros((_L,), jnp.int32)
        for c in range(1, C):
            v = _monokey(gt_v[pl.ds(c * P + off, _L)])
            m = v > best
            bidx = jnp.where(m, c, bidx)
            best = jnp.where(m, v, best)
        fov = fov_v[pl.ds(off, _L)]
        val = (fov == 1) & (bidx != _IGNORE)
        lab_v[pl.ds(off, _L)] = bidx
        val_v[pl.ds(off, _L)] = jnp.where(val, one, zero)
        for c in range(1, C):
            mi = jnp.where((bidx == c) & val, one, zero)
            hist_v[pl.ds(c * _L, _L)] = hist_v[pl.ds(c * _L, _L)] + mi
        return carry

    lax.fori_loop(0, nch, pA, 0)

    def pA2(_, carry):
        lanes = lax.iota(jnp.int32, _L)
        zero = jnp.zeros((_L,), jnp.int32)
        ca = zero
        cb = zero
        for c in range(1, C):
            acc = hist_v[pl.ds(c * _L, _L)]
            s = acc[0]
            for q in range(1, _L):
                s = s + acc[q]
            if c < _L:
                ca = jnp.where(lanes == c, zero + s, ca)
            else:
                cb = jnp.where(lanes == (c - _L), zero + s, cb)
        cnt_v[pl.ds(0, _L)] = ca
        cnt_v[pl.ds(_L, _L)] = cb
        return carry

    lax.fori_loop(0, 1, pA2, 0)

    pltpu.sync_copy(cnt_v, grid_sh.at[pl.ds(wid * 2 * _L, 2 * _L)])
    plsc.subcore_barrier()
    pltpu.sync_copy(grid_sh, grid_v)

    def pB(_, carry):
        lanes = lax.iota(jnp.int32, _L)
        zero = jnp.zeros((_L,), jnp.int32)
        ga = zero
        gb = zero
        pa = zero
        pb = zero
        for t in range(_NSUB):
            ra = grid_v[pl.ds(t * 2 * _L, _L)]
            rb = grid_v[pl.ds(t * 2 * _L + _L, _L)]
            ga = ga + ra
            gb = gb + rb
            cond = t < wid
            pa = pa + jnp.where(cond, ra, zero)
            pb = pb + jnp.where(cond, rb, zero)
        for c in range(1, C):
            pbase = pa[c] if c < _L else pb[c - _L]
            rbtab_v[pl.ds(c * _L, _L)] = zero + pbase

        g = [None] * C
        for c in range(1, C):
            g[c] = ga[c] if c < _L else gb[c - _L]

        kk = jnp.int32(0)
        for c in range(1, C):
            kk = kk + jnp.where(g[c] > 0, 1, 0)
        lo_t = lax.shift_right_arithmetic(kk - 1, 1)
        hi_t = lax.shift_right_arithmetic(kk, 1)
        lo = jnp.int32(0)
        hi = jnp.int32(0)
        for c in range(1, C):
            nzc = g[c] > 0
            rank = jnp.int32(0)
            for d in range(1, C):
                if d == c:
                    continue
                less = g[d] < g[c]
                tie = (g[d] == g[c]) & (d < c)
                rank = rank + jnp.where((g[d] > 0) & (less | tie), 1, 0)
            lo = jnp.where(nzc & (rank == lo_t), g[c], lo)
            hi = jnp.where(nzc & (rank == hi_t), g[c], hi)
        mcs = jnp.maximum(lax.shift_right_arithmetic(lo + hi, 1), _MAXPPC)

        inca = zero
        incb = zero
        sega = zero
        segb = zero
        run = jnp.int32(0)
        for c in range(1, C):
            selc = jnp.minimum(g[c], mcs)
            segc = run
            run = run + selc
            segtab_v[pl.ds(c * _L, _L)] = zero + segc
            inctab_v[pl.ds(c * _L, _L)] = zero + run
            if c < _L:
                sega = jnp.where(lanes == c, zero + segc, sega)
                inca = jnp.where(lanes == c, zero + run, inca)
            else:
                segb = jnp.where(lanes == (c - _L), zero + segc, segb)
                incb = jnp.where(lanes == (c - _L), zero + run, incb)
        incb = jnp.where(lanes > (C - 1 - _L), zero + run, incb)
        stat_v[pl.ds(_MC * _L, _L)] = zero + mcs
        inc_v[pl.ds(0, _L)] = inca
        inc_v[pl.ds(_L, _L)] = incb
        return carry

    lax.fori_loop(0, 1, pB, 0)

    @pl.when(wid == 0)
    def _():
        pltpu.sync_copy(inc_v, inc_hbm)

    def pC(i, carry):
        off = i * _L
        lanes = lax.iota(jnp.int32, _L)
        one = jnp.zeros((_L,), jnp.int32) + 1
        zero = jnp.zeros((_L,), jnp.int32)
        labv = lab_v[pl.ds(off, _L)]
        valv = val_v[pl.ds(off, _L)] == 1
        mcv = stat_v[pl.ds(_MC * _L, _L)]
        dst = zero + M
        labacc = one
        jvec = lanes + (base_px + off)
        for c in range(1, C):
            m = (labv == c) & valv
            mi = jnp.where(m, one, zero)
            x = mi
            for k in (1, 2, 4, 8):
                sh_v[pl.ds(_L, _L)] = x
                x = x + sh_v[pl.ds(_L - k, _L)]
            tot = x[_L - 1]
            prior = x - mi
            rbv = rbtab_v[pl.ds(c * _L, _L)]
            r = prior + rbv
            selm = m & (r < mcv)
            dst = jnp.where(selm, r + segtab_v[pl.ds(c * _L, _L)], dst)
            rbtab_v[pl.ds(c * _L, _L)] = rbv + tot
            labacc = labacc + jnp.where(
                jvec >= inctab_v[pl.ds(c * _L, _L)], one, zero)
        labo_v[pl.ds(off, _L)] = labacc
        dstil_v[pl.ds(off, _L)] = dst
        return carry

    lax.fori_loop(0, nch, pC, 0)
    pltpu.sync_copy(labo_v, labout_hbm.at[pl.ds(base_px, P)])

    def pD(q, carry):
        for j in range(6):
            dstN_v[j][pl.ds(q * _L, _L)] = (
                dstil_v[pl.ds(j * 96 + q * _L, _L)])
        return carry

    lax.fori_loop(0, 6, pD, 0)

    rows_desc.wait()
    descs = []
    for j in range(P // 96):
        descs.append(pltpu.async_copy(
            rows_v.at[pl.ds(j * 96, 96)], fsel_hbm.at[dstN_v[j]], sem))
    for d in descs:
        d.wait()


def _sc_select(gt1d, fov_i, feats_fl, C, M, HW):
    P = M // _NSUB
    Z = feats_fl.shape[1]
    mesh = plsc.VectorSubcoreMesh(core_axis_name="c", subcore_axis_name="s",
                                  num_cores=1)
    body = functools.partial(_sc_select_body, C, M, P, HW)
    return pl.kernel(
        body,
        out_type=(
            jax.ShapeDtypeStruct((M + 8, Z), jnp.float32),
            jax.ShapeDtypeStruct((M,), jnp.int32),
            jax.ShapeDtypeStruct((2 * _L,), jnp.int32),
        ),
        mesh=mesh,
        scratch_types=[
            pltpu.VMEM((C * P,), jnp.int32),
            pltpu.VMEM((P,), jnp.int32),
            pltpu.VMEM((P,), jnp.int32),
            pltpu.VMEM((P,), jnp.int32),
            pltpu.VMEM((2 * _L,), jnp.int32),
            pltpu.VMEM((20 * _L,), jnp.int32),
            pltpu.VMEM_SHARED((_NSUB * 2 * _L,), jnp.int32),
            pltpu.VMEM((_NSUB * 2 * _L,), jnp.int32),
            pltpu.VMEM((_NSTAT * _L,), jnp.int32),
            pltpu.VMEM((20 * _L,), jnp.int32),
            pltpu.VMEM((20 * _L,), jnp.int32),
            pltpu.VMEM((20 * _L,), jnp.int32),
            pltpu.VMEM((2 * _L,), jnp.int32),
            pltpu.VMEM((2 * _L,), jnp.int32),
            pltpu.VMEM((P,), jnp.int32),
            pltpu.VMEM((P,), jnp.int32),
            pltpu.VMEM((96,), jnp.int32),
            pltpu.VMEM((96,), jnp.int32),
            pltpu.VMEM((96,), jnp.int32),
            pltpu.VMEM((96,), jnp.int32),
            pltpu.VMEM((96,), jnp.int32),
            pltpu.VMEM((96,), jnp.int32),
            pltpu.VMEM((P, Z), jnp.float32),
            pltpu.SemaphoreType.DMA,
            pltpu.SemaphoreType.DMA,
        ],
    )(gt1d, fov_i, feats_fl)


def _supcon_body(s_ref, labr_ref, labc_ref, f_ref, out_ref, fn_ref, d_ref):
    S = s_ref[0]
    nb = (S + _BLK - 1) // _BLK
    inv_t = 1.0 / _TEMP
    shift = 1.0 / _TEMP

    def norm_body(rb, _):
        blk = f_ref[pl.ds(rb * _BLK, _BLK), :]
        n = jnp.sqrt(jnp.sum(blk * blk, axis=1, keepdims=True))
        fn_ref[pl.ds(rb * _BLK, _BLK), :] = blk / (n + 1e-12)
        return 0

    jax.lax.fori_loop(0, nb, norm_body, 0)

    def d_body(rb, _):
        labr = labr_ref[pl.ds(rb * _BLK, _BLK), :]
        a = fn_ref[pl.ds(rb * _BLK, _BLK), :]

        def cb_body(cb, acc):
            bblk = fn_ref[pl.ds(cb * _BLK, _BLK), :]
            s = jax.lax.dot_general(a, bblk, (((1,), (1,)), ((), ())),
                                    preferred_element_type=jnp.float32)
            l = s * inv_t - shift
            labc = labc_ref[:, pl.ds(cb * _BLK, _BLK)]
            colidx = jax.lax.broadcasted_iota(jnp.int32, (1, _BLK), 1) + cb * _BLK
            negm = (labr != labc) & (colidx < S)
            return acc + jnp.sum(jnp.where(negm, jnp.exp(l), 0.0), axis=1,
                                 keepdims=True)

        acc = jax.lax.fori_loop(0, nb, cb_body,
                                jnp.zeros((_BLK, 1), jnp.float32))
        d_ref[pl.ds(rb * _BLK, _BLK), :] = acc
        return 0

    jax.lax.fori_loop(0, nb, d_body, 0)

    def p_body(rb, carry):
        tot_p, tot_c = carry
        labr = labr_ref[pl.ds(rb * _BLK, _BLK), :]
        a = fn_ref[pl.ds(rb * _BLK, _BLK), :]
        dvec = d_ref[pl.ds(rb * _BLK, _BLK), :]
        rowidx = jax.lax.broadcasted_iota(jnp.int32, (_BLK, 1), 0) + rb * _BLK

        def cb_body(cb, carry2):
            psum, pcnt = carry2
            bblk = fn_ref[pl.ds(cb * _BLK, _BLK), :]
            s = jax.lax.dot_general(a, bblk, (((1,), (1,)), ((), ())),
                                    preferred_element_type=jnp.float32)
            labc = labc_ref[:, pl.ds(cb * _BLK, _BLK)]
            colidx = jax.lax.broadcasted_iota(jnp.int32, (_BLK, _BLK), 1) + cb * _BLK
            posm = (labr == labc) & (rowidx != colidx) & (colidx < S)
            l = jnp.minimum(s, _CLIP_POS) * inv_t - shift
            lp = l - jnp.log(jnp.exp(l) + dvec + 1e-12)
            psum = psum + jnp.sum(jnp.where(posm, lp, 0.0), axis=1,
                                  keepdims=True)
            pcnt = pcnt + jnp.sum(posm.astype(jnp.int32), axis=1,
                                  keepdims=True)
            return psum, pcnt

        psum, pcnt = jax.lax.fori_loop(
            0, nb, cb_body,
            (jnp.zeros((_BLK, 1), jnp.float32),
             jnp.zeros((_BLK, 1), jnp.int32)))
        haspos = pcnt > 0
        mlpp = psum / jnp.maximum(pcnt, 1).astype(jnp.float32)
        tot_p = tot_p + jnp.sum(jnp.where(haspos, mlpp, 0.0))
        tot_c = tot_c + jnp.sum(haspos.astype(jnp.int32))
        return tot_p, tot_c

    tot_p, tot_c = jax.lax.fori_loop(0, nb, p_body,
                                     (jnp.float32(0.0), jnp.int32(0)))
    loss = -tot_p / jnp.maximum(tot_c, 1).astype(jnp.float32)
    out_ref[0, 0] = _WEIGHT * loss


def _supcon_loss(f_sel, lab_sel, S, M):
    labr = lab_sel.reshape(M, 1)
    labc = lab_sel.reshape(1, M)
    out = pl.pallas_call(
        _supcon_body,
        out_shape=jax.ShapeDtypeStruct((1, 1), jnp.float32),
        in_specs=[
            pl.BlockSpec(memory_space=pltpu.SMEM),
            pl.BlockSpec(memory_space=pltpu.VMEM),
            pl.BlockSpec(memory_space=pltpu.VMEM),
            pl.BlockSpec(memory_space=pltpu.VMEM),
        ],
        out_specs=pl.BlockSpec(memory_space=pltpu.SMEM),
        scratch_shapes=[
            pltpu.VMEM((M, 128), jnp.float32),
            pltpu.VMEM((M, 1), jnp.float32),
        ],
    )(S, labr, labc, f_sel)
    return out[0, 0]


def kernel(feats, gt_prob, fov_mask):
    BV, Z, H, W = feats.shape
    B = BV // _VIEWS
    C = gt_prob.shape[1]
    M = B * H * W
    gt1d = lax.bitcast_convert_type(gt_prob, jnp.int32).reshape(BV * C * H * W)
    fov_i = fov_mask.reshape(M).astype(jnp.int32)
    feats_fl = feats.transpose(0, 2, 3, 1).reshape(M, Z)
    f_sel, lab_sel, inc = _sc_select(gt1d, fov_i, feats_fl, C, M, H * W)
    S = inc[C - 1:C]
    return _supcon_loss(f_sel, lab_sel, S, M)
